# baseline (speedup 1.0000x reference)
import jax
import jax.numpy as jnp
from jax.experimental import pallas as pl

_B = 16
_H = 12
_S = 577
_W = 640


def _body(a1_ref, a2_ref, out_ref):
    lane = jax.lax.broadcasted_iota(jnp.int32, (_B, _W), 1)
    valid = lane < _S

    def ranks(x):
        m = jnp.sum(x, axis=1) * (1.0 / _H)
        out = []
        for k in (19, 20, 21):
            vk = m[:, k:k + 1]
            less = jnp.where(valid & (m < vk), 1.0, 0.0)
            eq_before = jnp.where((m == vk) & (lane < k), 1.0, 0.0)
            out.append(jnp.sum(less + eq_before, axis=1, keepdims=True))
        return out

    r1 = ranks(a1_ref[...])
    r2 = ranks(a2_ref[...])
    lane3 = jax.lax.broadcasted_iota(jnp.int32, (_B, 128), 1)
    acc = jnp.zeros((_B, 128), jnp.float32)
    for i in range(3):
        v = (r1[i] + r2[i]) * 0.5 + 1.0
        acc = jnp.where(lane3 == i, v, acc)
    out_ref[...] = acc


def kernel(attn1, attn2):
    a1 = attn1.reshape(_B, _H, _S * _S)
    a2 = attn2.reshape(_B, _H, _S * _S)
    in_spec = pl.BlockSpec((_B, _H, _W), lambda i: (0, 0, 0))
    out = pl.pallas_call(
        _body,
        grid=(1,),
        in_specs=[in_spec, in_spec],
        out_specs=pl.BlockSpec((_B, 128), lambda i: (0, 0)),
        out_shape=jax.ShapeDtypeStruct((_B, 128), jnp.float32),
    )(a1, a2)
    return out[:, :3]


# speedup vs baseline: 2.8761x; 2.8761x over previous
import jax
import jax.numpy as jnp
from jax.experimental import pallas as pl

_B = 16
_H = 12
_S = 577
_W = 640


def _body(a1_ref, a2_ref, out_ref):
    lane = jax.lax.broadcasted_iota(jnp.int32, (_B, _W), 1)
    valid = lane < _S

    def ranks(x):
        m = jnp.sum(x, axis=1) * (1.0 / _H)
        out = []
        for k in (19, 20, 21):
            vk = m[:, k:k + 1]
            less = jnp.where(valid & (m < vk), 1.0, 0.0)
            eq_before = jnp.where((m == vk) & (lane < k), 1.0, 0.0)
            out.append(jnp.sum(less + eq_before, axis=1, keepdims=True))
        return out

    r1 = ranks(a1_ref[:, :, 0, :])
    r2 = ranks(a2_ref[:, :, 0, :])
    lane3 = jax.lax.broadcasted_iota(jnp.int32, (_B, 128), 1)
    acc = jnp.zeros((_B, 128), jnp.float32)
    for i in range(3):
        v = (r1[i] + r2[i]) * 0.5 + 1.0
        acc = jnp.where(lane3 == i, v, acc)
    out_ref[...] = acc


def kernel(attn1, attn2):
    in_spec = pl.BlockSpec((_B, _H, 8, _W), lambda i: (0, 0, 0, 0))
    out = pl.pallas_call(
        _body,
        grid=(1,),
        in_specs=[in_spec, in_spec],
        out_specs=pl.BlockSpec((_B, 128), lambda i: (0, 0)),
        out_shape=jax.ShapeDtypeStruct((_B, 128), jnp.float32),
    )(attn1, attn2)
    return out[:, :3]


# speedup vs baseline: 2.9206x; 1.0155x over previous
import jax
import jax.numpy as jnp
from jax.experimental import pallas as pl
from jax.experimental.pallas import tpu as pltpu

_B = 16
_H = 12
_S = 577


def _body(a1_ref, a2_ref, out_ref, s1, s2, sem):
    for b in range(_B):
        pltpu.make_async_copy(a1_ref.at[b, :, 0, :], s1.at[b], sem).start()
        pltpu.make_async_copy(a2_ref.at[b, :, 0, :], s2.at[b], sem).start()
    for b in range(_B):
        pltpu.make_async_copy(a1_ref.at[b, :, 0, :], s1.at[b], sem).wait()
        pltpu.make_async_copy(a2_ref.at[b, :, 0, :], s2.at[b], sem).wait()

    lane = jax.lax.broadcasted_iota(jnp.int32, (_B, _S), 1)

    def ranks(x):
        m = jnp.sum(x, axis=1) * (1.0 / _H)
        out = []
        for k in (19, 20, 21):
            vk = m[:, k:k + 1]
            less = jnp.where(m < vk, 1.0, 0.0)
            eq_before = jnp.where((m == vk) & (lane < k), 1.0, 0.0)
            out.append(jnp.sum(less + eq_before, axis=1, keepdims=True))
        return out

    r1 = ranks(s1[...])
    r2 = ranks(s2[...])
    lane3 = jax.lax.broadcasted_iota(jnp.int32, (_B, 128), 1)
    acc = jnp.zeros((_B, 128), jnp.float32)
    for i in range(3):
        v = (r1[i] + r2[i]) * 0.5 + 1.0
        acc = jnp.where(lane3 == i, v, acc)
    out_ref[...] = acc


def kernel(attn1, attn2):
    hbm_spec = pl.BlockSpec(memory_space=pltpu.MemorySpace.HBM)
    out = pl.pallas_call(
        _body,
        in_specs=[hbm_spec, hbm_spec],
        out_specs=pl.BlockSpec(memory_space=pltpu.MemorySpace.VMEM),
        out_shape=jax.ShapeDtypeStruct((_B, 128), jnp.float32),
        scratch_shapes=[
            pltpu.VMEM((_B, _H, _S), jnp.float32),
            pltpu.VMEM((_B, _H, _S), jnp.float32),
            pltpu.SemaphoreType.DMA,
        ],
    )(attn1, attn2)
    return out[:, :3]


# speedup vs baseline: 135.5530x; 46.4124x over previous
import jax
import jax.numpy as jnp
from jax.experimental import pallas as pl

_B = 16
_H = 12
_S = 577


def _body(x1_ref, x2_ref, out_ref):
    lane = jax.lax.broadcasted_iota(jnp.int32, (_B, _S), 1)

    def ranks(x):
        m = jnp.sum(x, axis=1) * (1.0 / _H)
        out = []
        for k in (19, 20, 21):
            vk = m[:, k:k + 1]
            less = jnp.where(m < vk, 1.0, 0.0)
            eq_before = jnp.where((m == vk) & (lane < k), 1.0, 0.0)
            out.append(jnp.sum(less + eq_before, axis=1, keepdims=True))
        return out

    r1 = ranks(x1_ref[...])
    r2 = ranks(x2_ref[...])
    lane3 = jax.lax.broadcasted_iota(jnp.int32, (_B, 128), 1)
    acc = jnp.zeros((_B, 128), jnp.float32)
    for i in range(3):
        v = (r1[i] + r2[i]) * 0.5 + 1.0
        acc = jnp.where(lane3 == i, v, acc)
    out_ref[...] = acc


def kernel(attn1, attn2):
    x1 = attn1[:, :, 0, :]
    x2 = attn2[:, :, 0, :]
    out = pl.pallas_call(
        _body,
        out_shape=jax.ShapeDtypeStruct((_B, 128), jnp.float32),
    )(x1, x2)
    return out[:, :3]
